# baseline (speedup 1.0000x reference)
import functools

import jax
import jax.numpy as jnp
from jax import lax
from jax.experimental import pallas as pl
from jax.experimental.pallas import tpu as pltpu
from jax.experimental.pallas import tpu_sc as plsc

E = 4096
D = 128
NREL = 64
BLK = 4096

_NC = 2
_NS = 16
_NW = _NC * _NS
_BPW = E // _NW


def _sc_gather(table, h_idx, t_idx):
  mesh = plsc.VectorSubcoreMesh(
      core_axis_name="c", subcore_axis_name="s",
      num_cores=_NC, num_subcores=_NS)

  @functools.partial(
      pl.kernel,
      out_type=(jax.ShapeDtypeStruct((E, D), jnp.float32),
                jax.ShapeDtypeStruct((E, D), jnp.float32)),
      mesh=mesh,
      scratch_types=(pltpu.VMEM((_BPW,), jnp.int32),
                     pltpu.VMEM((_BPW, D), jnp.float32),
                     pltpu.VMEM((_BPW,), jnp.int32),
                     pltpu.VMEM((_BPW, D), jnp.float32),
                     pltpu.SemaphoreType.DMA,
                     pltpu.SemaphoreType.DMA,
                     pltpu.SemaphoreType.DMA,
                     pltpu.SemaphoreType.DMA,
                     pltpu.SemaphoreType.DMA,
                     pltpu.SemaphoreType.DMA),
  )
  def k(table_hbm, hi_hbm, ti_hbm, h_out, t_out,
        hi_v, hrows_v, ti_v, trows_v,
        sem_ih, sem_it, sem_h, sem_t, sem_oh, sem_ot):
    wid = lax.axis_index("s") * _NC + lax.axis_index("c")
    base = wid * _BPW
    cih = pltpu.async_copy(hi_hbm.at[pl.ds(base, _BPW)], hi_v, sem_ih)
    cit = pltpu.async_copy(ti_hbm.at[pl.ds(base, _BPW)], ti_v, sem_it)
    cih.wait()
    ch = pltpu.async_copy(table_hbm.at[hi_v], hrows_v, sem_h)
    cit.wait()
    ct = pltpu.async_copy(table_hbm.at[ti_v], trows_v, sem_t)
    ch.wait()
    coh = pltpu.async_copy(hrows_v, h_out.at[pl.ds(base, _BPW)], sem_oh)
    ct.wait()
    cot = pltpu.async_copy(trows_v, t_out.at[pl.ds(base, _BPW)], sem_ot)
    coh.wait()
    cot.wait()

  return k(table, h_idx, t_idx)


def _tc_body(h_ref, t_ref, rel_ref, ridx_ref, w1h_ref, w2h_ref,
             w1r_ref, w2r_ref, bh_ref, br_ref, semw_ref, out_ref):
  mm = lambda a, b: jnp.dot(a, b, preferred_element_type=jnp.float32,
                            precision=lax.Precision.DEFAULT)
  H = h_ref[...]
  T = t_ref[...]
  R = rel_ref[...]
  Wt = semw_ref[:D, :]
  Wb = semw_ref[D:, :]
  RT = mm(R, Wt)
  RB = mm(R, Wb)
  sv1 = jnp.sum(R * w1h_ref[...], axis=1, keepdims=True)
  sv3 = jnp.sum(R * w2h_ref[...], axis=1, keepdims=True)
  sv13 = jnp.concatenate([sv1, sv3], axis=1)
  cvec = mm(bh_ref[...], Wt) + mm(br_ref[...], Wb)
  iota = lax.broadcasted_iota(jnp.int32, (BLK, NREL), 1)
  onehot = (iota == ridx_ref[...]).astype(jnp.float32)
  s13 = mm(onehot, sv13)
  s1 = s13[:, 0:1]
  s3 = s13[:, 1:2]
  s2 = jnp.sum(H * w2r_ref[...], axis=1, keepdims=True)
  s4 = jnp.sum(H * w1r_ref[...], axis=1, keepdims=True)
  proj = (mm(H * s1, Wt) + mm(H * s3, Wb)
          + mm(onehot * s2, RT) + mm(onehot * s4, RB) + cvec)
  x = proj * T
  y = jnp.where(x >= 0, x, 0.01 * x)
  out_ref[...] = jnp.sum(y, axis=1, keepdims=True)


def kernel(entity_embed, relation_embed, h_trans_w1, h_trans_w2, h_bias_b,
           r_trans_w1, r_trans_w2, r_bias_b, sem_trans_w,
           h_batch, t_batch, r_batch):
  H, T = _sc_gather(entity_embed, h_batch, t_batch)
  full = lambda shape: pl.BlockSpec(shape, lambda i: (0, 0))
  out = pl.pallas_call(
      _tc_body,
      grid=(E // BLK,),
      in_specs=[
          pl.BlockSpec((BLK, D), lambda i: (i, 0)),
          pl.BlockSpec((BLK, D), lambda i: (i, 0)),
          full((NREL, D)),
          pl.BlockSpec((BLK, 1), lambda i: (i, 0)),
          full((1, D)), full((1, D)),
          full((1, D)), full((1, D)),
          full((1, D)), full((1, D)),
          full((2 * D, D)),
      ],
      out_specs=pl.BlockSpec((BLK, 1), lambda i: (i, 0)),
      out_shape=jax.ShapeDtypeStruct((E, 1), jnp.float32),
  )(H, T, relation_embed, r_batch.reshape(E, 1),
    h_trans_w1.reshape(1, D), h_trans_w2.reshape(1, D),
    r_trans_w1.reshape(1, D), r_trans_w2.reshape(1, D),
    h_bias_b.reshape(1, D), r_bias_b.reshape(1, D), sem_trans_w)
  return out.reshape(E)


# speedup vs baseline: 1.0169x; 1.0169x over previous
import functools

import jax
import jax.numpy as jnp
from jax import lax
from jax.experimental import pallas as pl
from jax.experimental.pallas import tpu as pltpu
from jax.experimental.pallas import tpu_sc as plsc

E = 4096
D = 128
NREL = 64
BLK = 2048

_NC = 2
_NS = 16
_NW = _NC * _NS
_BPW = E // _NW


def _sc_gather(table, h_idx, t_idx):
  mesh = plsc.VectorSubcoreMesh(
      core_axis_name="c", subcore_axis_name="s",
      num_cores=_NC, num_subcores=_NS)

  @functools.partial(
      pl.kernel,
      out_type=(jax.ShapeDtypeStruct((E, D), jnp.float32),
                jax.ShapeDtypeStruct((E, D), jnp.float32)),
      mesh=mesh,
      scratch_types=(pltpu.VMEM((_BPW,), jnp.int32),
                     pltpu.VMEM((_BPW, D), jnp.float32),
                     pltpu.VMEM((_BPW,), jnp.int32),
                     pltpu.VMEM((_BPW, D), jnp.float32),
                     pltpu.SemaphoreType.DMA,
                     pltpu.SemaphoreType.DMA,
                     pltpu.SemaphoreType.DMA,
                     pltpu.SemaphoreType.DMA,
                     pltpu.SemaphoreType.DMA,
                     pltpu.SemaphoreType.DMA,
                     pltpu.SemaphoreType.DMA,
                     pltpu.SemaphoreType.DMA,
                     pltpu.SemaphoreType.DMA,
                     pltpu.SemaphoreType.DMA),
  )
  def k(table_hbm, hi_hbm, ti_hbm, h_out, t_out,
        hi_v, hrows_v, ti_v, trows_v,
        sem_ih, sem_it, sem_h0, sem_h1, sem_t0, sem_t1,
        sem_oh0, sem_oh1, sem_ot0, sem_ot1):
    wid = lax.axis_index("s") * _NC + lax.axis_index("c")
    base = wid * _BPW
    half = _BPW // 2
    cih = pltpu.async_copy(hi_hbm.at[pl.ds(base, _BPW)], hi_v, sem_ih)
    cit = pltpu.async_copy(ti_hbm.at[pl.ds(base, _BPW)], ti_v, sem_it)
    cih.wait()
    ch0 = pltpu.async_copy(table_hbm.at[hi_v.at[pl.ds(0, half)]],
                           hrows_v.at[pl.ds(0, half)], sem_h0)
    ch1 = pltpu.async_copy(table_hbm.at[hi_v.at[pl.ds(half, half)]],
                           hrows_v.at[pl.ds(half, half)], sem_h1)
    cit.wait()
    ct0 = pltpu.async_copy(table_hbm.at[ti_v.at[pl.ds(0, half)]],
                           trows_v.at[pl.ds(0, half)], sem_t0)
    ct1 = pltpu.async_copy(table_hbm.at[ti_v.at[pl.ds(half, half)]],
                           trows_v.at[pl.ds(half, half)], sem_t1)
    ch0.wait()
    coh0 = pltpu.async_copy(hrows_v.at[pl.ds(0, half)],
                            h_out.at[pl.ds(base, half)], sem_oh0)
    ch1.wait()
    coh1 = pltpu.async_copy(hrows_v.at[pl.ds(half, half)],
                            h_out.at[pl.ds(base + half, half)], sem_oh1)
    ct0.wait()
    cot0 = pltpu.async_copy(trows_v.at[pl.ds(0, half)],
                            t_out.at[pl.ds(base, half)], sem_ot0)
    ct1.wait()
    cot1 = pltpu.async_copy(trows_v.at[pl.ds(half, half)],
                            t_out.at[pl.ds(base + half, half)], sem_ot1)
    coh0.wait()
    coh1.wait()
    cot0.wait()
    cot1.wait()

  return k(table, h_idx, t_idx)


def _tc_body(h_ref, t_ref, rel_ref, ridx_ref, w1h_ref, w2h_ref,
             w1r_ref, w2r_ref, bh_ref, br_ref, semw_ref, out_ref):
  mm = lambda a, b: jnp.dot(a, b, preferred_element_type=jnp.float32,
                            precision=lax.Precision.DEFAULT)
  H = h_ref[...]
  T = t_ref[...]
  R = rel_ref[...]
  Wt = semw_ref[:D, :]
  Wb = semw_ref[D:, :]
  RT = mm(R, Wt)
  RB = mm(R, Wb)
  sv1 = jnp.sum(R * w1h_ref[...], axis=1, keepdims=True)
  sv3 = jnp.sum(R * w2h_ref[...], axis=1, keepdims=True)
  sv13 = jnp.concatenate([sv1, sv3], axis=1)
  cvec = mm(bh_ref[...], Wt) + mm(br_ref[...], Wb)
  iota = lax.broadcasted_iota(jnp.int32, (BLK, NREL), 1)
  onehot = (iota == ridx_ref[...]).astype(jnp.float32)
  s13 = mm(onehot, sv13)
  s1 = s13[:, 0:1]
  s3 = s13[:, 1:2]
  s2 = jnp.sum(H * w2r_ref[...], axis=1, keepdims=True)
  s4 = jnp.sum(H * w1r_ref[...], axis=1, keepdims=True)
  proj = (mm(H * s1, Wt) + mm(H * s3, Wb)
          + mm(onehot * s2, RT) + mm(onehot * s4, RB) + cvec)
  x = proj * T
  y = jnp.where(x >= 0, x, 0.01 * x)
  out_ref[...] = jnp.sum(y, axis=1, keepdims=True)


def kernel(entity_embed, relation_embed, h_trans_w1, h_trans_w2, h_bias_b,
           r_trans_w1, r_trans_w2, r_bias_b, sem_trans_w,
           h_batch, t_batch, r_batch):
  H, T = _sc_gather(entity_embed, h_batch, t_batch)
  full = lambda shape: pl.BlockSpec(shape, lambda i: (0, 0))
  out = pl.pallas_call(
      _tc_body,
      grid=(E // BLK,),
      in_specs=[
          pl.BlockSpec((BLK, D), lambda i: (i, 0)),
          pl.BlockSpec((BLK, D), lambda i: (i, 0)),
          full((NREL, D)),
          pl.BlockSpec((BLK, 1), lambda i: (i, 0)),
          full((1, D)), full((1, D)),
          full((1, D)), full((1, D)),
          full((1, D)), full((1, D)),
          full((2 * D, D)),
      ],
      out_specs=pl.BlockSpec((BLK, 1), lambda i: (i, 0)),
      out_shape=jax.ShapeDtypeStruct((E, 1), jnp.float32),
  )(H, T, relation_embed, r_batch.reshape(E, 1),
    h_trans_w1.reshape(1, D), h_trans_w2.reshape(1, D),
    r_trans_w1.reshape(1, D), r_trans_w2.reshape(1, D),
    h_bias_b.reshape(1, D), r_bias_b.reshape(1, D), sem_trans_w)
  return out.reshape(E)


# speedup vs baseline: 1.0253x; 1.0082x over previous
import functools

import jax
import jax.numpy as jnp
from jax import lax
from jax.experimental import pallas as pl
from jax.experimental.pallas import tpu as pltpu
from jax.experimental.pallas import tpu_sc as plsc

E = 4096
D = 128
NREL = 64
BLK = 2048

_NC = 2
_NS = 16
_NW = _NC * _NS
_BPW = E // _NW


def _sc_gather(table, h_idx, t_idx):
  mesh = plsc.VectorSubcoreMesh(
      core_axis_name="c", subcore_axis_name="s",
      num_cores=_NC, num_subcores=_NS)

  @functools.partial(
      pl.kernel,
      out_type=(jax.ShapeDtypeStruct((E, D), jnp.float32),
                jax.ShapeDtypeStruct((E, D), jnp.float32)),
      mesh=mesh,
      scratch_types=(pltpu.VMEM((_BPW,), jnp.int32),
                     pltpu.VMEM((_BPW, D), jnp.float32),
                     pltpu.VMEM((_BPW,), jnp.int32),
                     pltpu.VMEM((_BPW, D), jnp.float32),
                     pltpu.SemaphoreType.DMA,
                     pltpu.SemaphoreType.DMA,
                     pltpu.SemaphoreType.DMA,
                     pltpu.SemaphoreType.DMA,
                     pltpu.SemaphoreType.DMA,
                     pltpu.SemaphoreType.DMA),
  )
  def k(table_hbm, hi_hbm, ti_hbm, h_out, t_out,
        hi_v, hrows_v, ti_v, trows_v,
        sem_ih, sem_it, sem_h, sem_t, sem_oh, sem_ot):
    wid = lax.axis_index("s") * _NC + lax.axis_index("c")
    base = wid * _BPW
    cih = pltpu.async_copy(hi_hbm.at[pl.ds(base, _BPW)], hi_v, sem_ih)
    cit = pltpu.async_copy(ti_hbm.at[pl.ds(base, _BPW)], ti_v, sem_it)
    cih.wait()
    ch = pltpu.async_copy(table_hbm.at[hi_v], hrows_v, sem_h)
    cit.wait()
    ct = pltpu.async_copy(table_hbm.at[ti_v], trows_v, sem_t)
    ch.wait()
    coh = pltpu.async_copy(hrows_v, h_out.at[pl.ds(base, _BPW)], sem_oh)
    ct.wait()
    cot = pltpu.async_copy(trows_v, t_out.at[pl.ds(base, _BPW)], sem_ot)
    coh.wait()
    cot.wait()

  return k(table, h_idx, t_idx)


def _tc_body(h_ref, t_ref, rel_ref, ridx_ref, w13_ref, w24_ref,
             bh_ref, br_ref, semw_ref, out_ref):
  mm = lambda a, b: jnp.dot(a, b, preferred_element_type=jnp.float32,
                            precision=lax.Precision.DEFAULT)
  H = h_ref[...]
  T = t_ref[...]
  R = rel_ref[...]
  Wt = semw_ref[:D, :]
  Wb = semw_ref[D:, :]
  RT = mm(R, Wt)
  RB = mm(R, Wb)
  sv13 = mm(R, w13_ref[...])
  cvec = mm(bh_ref[...], Wt) + mm(br_ref[...], Wb)
  iota = lax.broadcasted_iota(jnp.int32, (BLK, NREL), 1)
  onehot = (iota == ridx_ref[...]).astype(jnp.float32)
  s13 = mm(onehot, sv13)
  s1 = s13[:, 0:1]
  s3 = s13[:, 1:2]
  s24 = mm(H, w24_ref[...])
  s2 = s24[:, 0:1]
  s4 = s24[:, 1:2]
  proj = (mm(H * s1, Wt) + mm(H * s3, Wb)
          + mm(onehot * s2, RT) + mm(onehot * s4, RB) + cvec)
  x = proj * T
  y = jnp.where(x >= 0, x, 0.01 * x)
  out_ref[...] = jnp.sum(y, axis=1, keepdims=True)


def kernel(entity_embed, relation_embed, h_trans_w1, h_trans_w2, h_bias_b,
           r_trans_w1, r_trans_w2, r_bias_b, sem_trans_w,
           h_batch, t_batch, r_batch):
  H, T = _sc_gather(entity_embed, h_batch, t_batch)
  full = lambda shape: pl.BlockSpec(shape, lambda i: (0, 0))
  out = pl.pallas_call(
      _tc_body,
      grid=(E // BLK,),
      in_specs=[
          pl.BlockSpec((BLK, D), lambda i: (i, 0)),
          pl.BlockSpec((BLK, D), lambda i: (i, 0)),
          full((NREL, D)),
          pl.BlockSpec((BLK, 1), lambda i: (i, 0)),
          full((D, 2)),
          full((D, 2)),
          full((1, D)), full((1, D)),
          full((2 * D, D)),
      ],
      out_specs=pl.BlockSpec((BLK, 1), lambda i: (i, 0)),
      out_shape=jax.ShapeDtypeStruct((E, 1), jnp.float32),
  )(H, T, relation_embed, r_batch.reshape(E, 1),
    jnp.stack([h_trans_w1, h_trans_w2], axis=1),
    jnp.stack([r_trans_w2, r_trans_w1], axis=1),
    h_bias_b.reshape(1, D), r_bias_b.reshape(1, D), sem_trans_w)
  return out.reshape(E)


# speedup vs baseline: 1.0263x; 1.0010x over previous
import functools

import jax
import jax.numpy as jnp
from jax import lax
from jax.experimental import pallas as pl
from jax.experimental.pallas import tpu as pltpu
from jax.experimental.pallas import tpu_sc as plsc

E = 4096
D = 128
NREL = 64
BLK = 2048

_NC = 2
_NS = 16
_NW = _NC * _NS
_BPW = E // _NW


def _sc_gather(table, h_idx, t_idx):
  mesh = plsc.VectorSubcoreMesh(
      core_axis_name="c", subcore_axis_name="s",
      num_cores=_NC, num_subcores=_NS)

  @functools.partial(
      pl.kernel,
      out_type=(jax.ShapeDtypeStruct((E, D), jnp.float32),
                jax.ShapeDtypeStruct((E, D), jnp.float32)),
      mesh=mesh,
      scratch_types=(pltpu.VMEM((_BPW,), jnp.int32),
                     pltpu.VMEM((_BPW, D), jnp.float32),
                     pltpu.VMEM((_BPW,), jnp.int32),
                     pltpu.VMEM((_BPW, D), jnp.float32),
                     pltpu.SemaphoreType.DMA,
                     pltpu.SemaphoreType.DMA,
                     pltpu.SemaphoreType.DMA,
                     pltpu.SemaphoreType.DMA,
                     pltpu.SemaphoreType.DMA,
                     pltpu.SemaphoreType.DMA),
  )
  def k(table_hbm, hi_hbm, ti_hbm, h_out, t_out,
        hi_v, hrows_v, ti_v, trows_v,
        sem_ih, sem_it, sem_h, sem_t, sem_oh, sem_ot):
    wid = lax.axis_index("s") * _NC + lax.axis_index("c")
    base = wid * _BPW
    cih = pltpu.async_copy(hi_hbm.at[pl.ds(base, _BPW)], hi_v, sem_ih)
    cit = pltpu.async_copy(ti_hbm.at[pl.ds(base, _BPW)], ti_v, sem_it)
    cih.wait()
    ch = pltpu.async_copy(table_hbm.at[hi_v], hrows_v, sem_h)
    cit.wait()
    ct = pltpu.async_copy(table_hbm.at[ti_v], trows_v, sem_t)
    ch.wait()
    coh = pltpu.async_copy(hrows_v, h_out.at[pl.ds(base, _BPW)], sem_oh)
    ct.wait()
    cot = pltpu.async_copy(trows_v, t_out.at[pl.ds(base, _BPW)], sem_ot)
    coh.wait()
    cot.wait()

  return k(table, h_idx, t_idx)


def _tc_body(h_ref, t_ref, rel_ref, ridx_ref, w13_ref, w24_ref,
             bh_ref, br_ref, semw_ref, out_ref):
  mm = lambda a, b: jnp.dot(a, b, preferred_element_type=jnp.float32,
                            precision=lax.Precision.DEFAULT)
  H = h_ref[...]
  T = t_ref[...]
  R = rel_ref[...]
  Wt = semw_ref[:D, :]
  Wb = semw_ref[D:, :]
  RT = mm(R, Wt)
  RB = mm(R, Wb)
  sv13 = mm(R, w13_ref[...])
  sv1b = jnp.broadcast_to(sv13[:, 0:1], (NREL, D))
  sv3b = jnp.broadcast_to(sv13[:, 1:2], (NREL, D))
  cvec = mm(bh_ref[...], Wt) + mm(br_ref[...], Wb)
  iota = lax.broadcasted_iota(jnp.int32, (BLK, NREL), 1)
  onehot = (iota == ridx_ref[...]).astype(jnp.float32)
  s1 = mm(onehot, sv1b)
  s3 = mm(onehot, sv3b)
  s24 = mm(H, w24_ref[...])
  s2 = s24[:, 0:1]
  s4 = s24[:, 1:2]
  proj = (mm(H * s1, Wt) + mm(H * s3, Wb)
          + mm(onehot * s2, RT) + mm(onehot * s4, RB) + cvec)
  x = proj * T
  y = jnp.where(x >= 0, x, 0.01 * x)
  out_ref[...] = jnp.sum(y, axis=1, keepdims=True)


def kernel(entity_embed, relation_embed, h_trans_w1, h_trans_w2, h_bias_b,
           r_trans_w1, r_trans_w2, r_bias_b, sem_trans_w,
           h_batch, t_batch, r_batch):
  H, T = _sc_gather(entity_embed, h_batch, t_batch)
  full = lambda shape: pl.BlockSpec(shape, lambda i: (0, 0))
  out = pl.pallas_call(
      _tc_body,
      grid=(E // BLK,),
      in_specs=[
          pl.BlockSpec((BLK, D), lambda i: (i, 0)),
          pl.BlockSpec((BLK, D), lambda i: (i, 0)),
          full((NREL, D)),
          pl.BlockSpec((BLK, 1), lambda i: (i, 0)),
          full((D, 2)),
          full((D, 2)),
          full((1, D)), full((1, D)),
          full((2 * D, D)),
      ],
      out_specs=pl.BlockSpec((BLK, 1), lambda i: (i, 0)),
      out_shape=jax.ShapeDtypeStruct((E, 1), jnp.float32),
  )(H, T, relation_embed, r_batch.reshape(E, 1),
    jnp.stack([h_trans_w1, h_trans_w2], axis=1),
    jnp.stack([r_trans_w2, r_trans_w1], axis=1),
    h_bias_b.reshape(1, D), r_bias_b.reshape(1, D), sem_trans_w)
  return out.reshape(E)
